# baseline (speedup 1.0000x reference)
import jax
import jax.numpy as jnp
from jax import lax
from jax.experimental import pallas as pl


def _loss_block(logits_ref, cls_ref, pb_ref, tb_ref, out_ref):
    b = pl.program_id(0)

    @pl.when(b == 0)
    def _init():
        out_ref[...] = jnp.zeros_like(out_ref)

    x = logits_ref[...]
    cls3 = cls_ref[...]
    q, c = x.shape[1], x.shape[2]

    lane = lax.broadcasted_iota(jnp.int32, (1, q, c), 2)
    t = (lane == cls3).astype(jnp.float32)

    prob = jax.nn.sigmoid(x)
    ce = jnp.maximum(x, 0.0) - x * t + jnp.log1p(jnp.exp(-jnp.abs(x)))
    p_t = prob * t + (1.0 - prob) * (1.0 - t)
    one_m_pt = 1.0 - p_t
    loss = (0.25 * t + 0.75 * (1.0 - t)) * ce * one_m_pt * one_m_pt
    s_ce = jnp.sum(loss)

    matched3 = (cls3 != c).astype(jnp.float32)
    s_match = jnp.sum(matched3)

    pb = pb_ref[...]
    tb = tb_ref[...]
    s_l1 = jnp.sum(jnp.sum(jnp.abs(pb - tb), axis=-1, keepdims=True) * matched3)

    def corners(bx):
        cx = bx[..., 0:1]
        cy = bx[..., 1:2]
        w = bx[..., 2:3]
        h = bx[..., 3:4]
        return cx - 0.5 * w, cy - 0.5 * h, cx + 0.5 * w, cy + 0.5 * h

    ax0, ay0, ax1, ay1 = corners(pb)
    bx0, by0, bx1, by1 = corners(tb)
    area_a = (ax1 - ax0) * (ay1 - ay0)
    area_b = (bx1 - bx0) * (by1 - by0)
    iw = jnp.clip(jnp.minimum(ax1, bx1) - jnp.maximum(ax0, bx0), 0.0)
    ih = jnp.clip(jnp.minimum(ay1, by1) - jnp.maximum(ay0, by0), 0.0)
    inter = iw * ih
    union = area_a + area_b - inter
    iou = inter / (union + 1e-7)
    ew = jnp.clip(jnp.maximum(ax1, bx1) - jnp.minimum(ax0, bx0), 0.0)
    eh = jnp.clip(jnp.maximum(ay1, by1) - jnp.minimum(ay0, by0), 0.0)
    area_e = ew * eh
    giou = iou - (area_e - union) / (area_e + 1e-7)
    s_giou = jnp.sum((1.0 - giou) * matched3)

    lane128 = lax.broadcasted_iota(jnp.int32, (1, 128), 1)
    acc = (s_ce * (lane128 == 0) + s_match * (lane128 == 1)
           + s_l1 * (lane128 == 2) + s_giou * (lane128 == 3))
    out_ref[...] += acc


def kernel(logits, pred_boxes, target_boxes, target_classes):
    B, Q, C = logits.shape
    cls = target_classes.astype(jnp.int32).reshape(B, Q, 1)
    sums = pl.pallas_call(
        _loss_block,
        grid=(B,),
        in_specs=[
            pl.BlockSpec((1, Q, C), lambda b: (b, 0, 0)),
            pl.BlockSpec((1, Q, 1), lambda b: (b, 0, 0)),
            pl.BlockSpec((1, Q, 4), lambda b: (b, 0, 0)),
            pl.BlockSpec((1, Q, 4), lambda b: (b, 0, 0)),
        ],
        out_specs=pl.BlockSpec((1, 128), lambda b: (0, 0)),
        out_shape=jax.ShapeDtypeStruct((1, 128), jnp.float32),
    )(logits, cls, pred_boxes, target_boxes)
    s = sums[0]
    num_boxes = jnp.maximum(s[1], 1.0)
    return jnp.stack([s[0] / num_boxes, s[2] / num_boxes, s[3] / num_boxes])


# speedup vs baseline: 1.7892x; 1.7892x over previous
import jax
import jax.numpy as jnp
from jax import lax
from jax.experimental import pallas as pl

_BB = 2


def _loss_block(logits_ref, clsa_ref, clsb_ref, pb_ref, tb_ref,
                ce_acc, m_acc, l1_acc, gi_acc, out_ref):
    b = pl.program_id(0)
    nb = pl.num_programs(0)

    @pl.when(b == 0)
    def _init():
        ce_acc[...] = jnp.zeros_like(ce_acc)
        m_acc[...] = jnp.zeros_like(m_acc)
        l1_acc[...] = jnp.zeros_like(l1_acc)
        gi_acc[...] = jnp.zeros_like(gi_acc)

    x = logits_ref[...]
    cls3 = clsa_ref[...]
    bb, q, c = x.shape

    lane = lax.broadcasted_iota(jnp.int32, (bb, q, c), 2)
    tb = lane == cls3
    y = jnp.where(tb, -x, x)
    ax = jnp.abs(x)
    e = jnp.exp(-ax)
    lp = jnp.log1p(e)
    s = 1.0 / (1.0 + e)
    sp = jnp.maximum(y, 0.0) + lp
    sg = jnp.where(y >= 0.0, s, 1.0 - s)
    alpha = jnp.where(tb, 0.25, 0.75)
    loss = alpha * sp * sg * sg
    ce_acc[...] += jnp.sum(loss.reshape(bb * q // 8, 8, c), axis=0)

    clsb = clsb_ref[...]
    matched = (clsb != c).astype(jnp.float32)
    pb = pb_ref[...]
    tbx = tb_ref[...]
    l1 = jnp.sum(jnp.abs(pb - tbx), axis=1, keepdims=True) * matched

    def corners(bx):
        cx = bx[:, 0:1, :]
        cy = bx[:, 1:2, :]
        w = bx[:, 2:3, :]
        h = bx[:, 3:4, :]
        return cx - 0.5 * w, cy - 0.5 * h, cx + 0.5 * w, cy + 0.5 * h

    ax0, ay0, ax1, ay1 = corners(pb)
    bx0, by0, bx1, by1 = corners(tbx)
    area_a = (ax1 - ax0) * (ay1 - ay0)
    area_b = (bx1 - bx0) * (by1 - by0)
    iw = jnp.maximum(jnp.minimum(ax1, bx1) - jnp.maximum(ax0, bx0), 0.0)
    ih = jnp.maximum(jnp.minimum(ay1, by1) - jnp.maximum(ay0, by0), 0.0)
    inter = iw * ih
    union = area_a + area_b - inter
    iou = inter / (union + 1e-7)
    ew = jnp.maximum(jnp.maximum(ax1, bx1) - jnp.minimum(ax0, bx0), 0.0)
    eh = jnp.maximum(jnp.maximum(ay1, by1) - jnp.minimum(ay0, by0), 0.0)
    area_e = ew * eh
    giou = iou - (area_e - union) / (area_e + 1e-7)
    gi = (1.0 - giou) * matched

    m_acc[...] += jnp.sum(matched, axis=0)
    l1_acc[...] += jnp.sum(l1, axis=0)
    gi_acc[...] += jnp.sum(gi, axis=0)

    @pl.when(b == nb - 1)
    def _final():
        s_ce = jnp.sum(ce_acc[...])
        s_m = jnp.sum(m_acc[...])
        s_l1 = jnp.sum(l1_acc[...])
        s_gi = jnp.sum(gi_acc[...])
        lane128 = lax.broadcasted_iota(jnp.int32, (1, 128), 1)
        out_ref[...] = (s_ce * (lane128 == 0) + s_m * (lane128 == 1)
                        + s_l1 * (lane128 == 2) + s_gi * (lane128 == 3))


def kernel(logits, pred_boxes, target_boxes, target_classes):
    B, Q, C = logits.shape
    cls = target_classes.astype(jnp.int32)
    cls_a = cls.reshape(B, Q, 1)
    cls_b = cls.reshape(B, 1, Q)
    pbt = pred_boxes.transpose(0, 2, 1)
    tbt = target_boxes.transpose(0, 2, 1)
    grid = B // _BB
    outs = pl.pallas_call(
        _loss_block,
        grid=(grid,),
        in_specs=[
            pl.BlockSpec((_BB, Q, C), lambda b: (b, 0, 0)),
            pl.BlockSpec((_BB, Q, 1), lambda b: (b, 0, 0)),
            pl.BlockSpec((_BB, 1, Q), lambda b: (b, 0, 0)),
            pl.BlockSpec((_BB, 4, Q), lambda b: (b, 0, 0)),
            pl.BlockSpec((_BB, 4, Q), lambda b: (b, 0, 0)),
        ],
        out_specs=[
            pl.BlockSpec((8, C), lambda b: (0, 0)),
            pl.BlockSpec((1, Q), lambda b: (0, 0)),
            pl.BlockSpec((1, Q), lambda b: (0, 0)),
            pl.BlockSpec((1, Q), lambda b: (0, 0)),
            pl.BlockSpec((1, 128), lambda b: (0, 0)),
        ],
        out_shape=[
            jax.ShapeDtypeStruct((8, C), jnp.float32),
            jax.ShapeDtypeStruct((1, Q), jnp.float32),
            jax.ShapeDtypeStruct((1, Q), jnp.float32),
            jax.ShapeDtypeStruct((1, Q), jnp.float32),
            jax.ShapeDtypeStruct((1, 128), jnp.float32),
        ],
    )(logits, cls_a, cls_b, pbt, tbt)
    s = outs[4][0]
    num_boxes = jnp.maximum(s[1], 1.0)
    return jnp.stack([s[0] / num_boxes, s[2] / num_boxes, s[3] / num_boxes])


# speedup vs baseline: 2.1312x; 1.1912x over previous
import jax
import jax.numpy as jnp
from jax import lax
from jax.experimental import pallas as pl

_BB = 8


def _loss_block(logits_ref, cls_ref, pb_ref, tb_ref,
                ce_acc, box_acc, out_ref):
    b = pl.program_id(0)
    nb = pl.num_programs(0)

    @pl.when(b == 0)
    def _init():
        ce_acc[...] = jnp.zeros_like(ce_acc)
        box_acc[...] = jnp.zeros_like(box_acc)

    x = logits_ref[...]
    cls = cls_ref[...]
    bb, q, c = x.shape
    cls3 = cls.reshape(bb, q, 1)

    lane = lax.broadcasted_iota(jnp.int32, (bb, q, c), 2)
    tb = lane == cls3
    y = jnp.where(tb, -x, x)
    e = jnp.exp(-jnp.abs(x))
    lp = jnp.log1p(e)
    s = 1.0 / (1.0 + e)
    sp = jnp.maximum(y, 0.0) + lp
    sg = jnp.where(y >= 0.0, s, 1.0 - s)
    alpha = jnp.where(tb, 0.25, 0.75)
    loss = alpha * sp * sg * sg
    ce_acc[...] += jnp.sum(loss.reshape(bb * q // 8, 8, c), axis=0)

    matched = (cls.reshape(bb, 1, q) != c).astype(jnp.float32)
    pb = jnp.swapaxes(pb_ref[...], 1, 2)
    tbx = jnp.swapaxes(tb_ref[...], 1, 2)
    l1 = jnp.sum(jnp.abs(pb - tbx), axis=1, keepdims=True) * matched

    def corners(bx):
        cx = bx[:, 0:1, :]
        cy = bx[:, 1:2, :]
        w = bx[:, 2:3, :]
        h = bx[:, 3:4, :]
        return cx - 0.5 * w, cy - 0.5 * h, cx + 0.5 * w, cy + 0.5 * h

    ax0, ay0, ax1, ay1 = corners(pb)
    bx0, by0, bx1, by1 = corners(tbx)
    area_a = (ax1 - ax0) * (ay1 - ay0)
    area_b = (bx1 - bx0) * (by1 - by0)
    iw = jnp.maximum(jnp.minimum(ax1, bx1) - jnp.maximum(ax0, bx0), 0.0)
    ih = jnp.maximum(jnp.minimum(ay1, by1) - jnp.maximum(ay0, by0), 0.0)
    inter = iw * ih
    union = area_a + area_b - inter
    iou = inter / (union + 1e-7)
    ew = jnp.maximum(jnp.maximum(ax1, bx1) - jnp.minimum(ax0, bx0), 0.0)
    eh = jnp.maximum(jnp.maximum(ay1, by1) - jnp.minimum(ay0, by0), 0.0)
    area_e = ew * eh
    giou = iou - (area_e - union) / (area_e + 1e-7)
    gi = (1.0 - giou) * matched

    box_acc[...] += jnp.concatenate(
        [jnp.sum(matched, axis=0), jnp.sum(l1, axis=0), jnp.sum(gi, axis=0)],
        axis=0)

    @pl.when(b == nb - 1)
    def _final():
        s_ce = jnp.sum(ce_acc[...])
        sums = jnp.sum(box_acc[...], axis=1)
        num_boxes = jnp.maximum(sums[0], 1.0)
        lane128 = lax.broadcasted_iota(jnp.int32, (1, 128), 1)
        out_ref[...] = (s_ce * (lane128 == 0) + sums[1] * (lane128 == 1)
                        + sums[2] * (lane128 == 2)) / num_boxes


def kernel(logits, pred_boxes, target_boxes, target_classes):
    B, Q, C = logits.shape
    cls = target_classes.astype(jnp.int32)
    grid = B // _BB
    outs = pl.pallas_call(
        _loss_block,
        grid=(grid,),
        in_specs=[
            pl.BlockSpec((_BB, Q, C), lambda b: (b, 0, 0)),
            pl.BlockSpec((_BB, Q), lambda b: (b, 0)),
            pl.BlockSpec((_BB, Q, 4), lambda b: (b, 0, 0)),
            pl.BlockSpec((_BB, Q, 4), lambda b: (b, 0, 0)),
        ],
        out_specs=[
            pl.BlockSpec((8, C), lambda b: (0, 0)),
            pl.BlockSpec((3, Q), lambda b: (0, 0)),
            pl.BlockSpec((1, 128), lambda b: (0, 0)),
        ],
        out_shape=[
            jax.ShapeDtypeStruct((8, C), jnp.float32),
            jax.ShapeDtypeStruct((3, Q), jnp.float32),
            jax.ShapeDtypeStruct((1, 128), jnp.float32),
        ],
    )(logits, cls, pred_boxes, target_boxes)
    return outs[2][0, :3]


# speedup vs baseline: 5.0688x; 2.3784x over previous
import jax
import jax.numpy as jnp
from jax import lax
from jax.experimental import pallas as pl

_BB = 8


def _loss_block(x_ref, cls_ref, pb_ref, tb_ref,
                ce_acc, m_acc, l1_acc, gi_acc, out_ref):
    b = pl.program_id(0)
    nb = pl.num_programs(0)

    @pl.when(b == 0)
    def _init():
        ce_acc[...] = jnp.zeros_like(ce_acc)
        m_acc[...] = jnp.zeros_like(m_acc)
        l1_acc[...] = jnp.zeros_like(l1_acc)
        gi_acc[...] = jnp.zeros_like(gi_acc)

    x = x_ref[...]
    cls = cls_ref[...]
    c = x.shape[0]

    cidx = lax.broadcasted_iota(jnp.int32, x.shape, 0)
    t = cidx == cls[None, :, :]
    y = jnp.where(t, -x, x)
    e = jnp.exp(-jnp.abs(x))
    lp = jnp.log1p(e)
    s = 1.0 / (1.0 + e)
    sp = jnp.maximum(y, 0.0) + lp
    sg = jnp.where(y >= 0.0, s, 1.0 - s)
    alpha = jnp.where(t, 0.25, 0.75)
    loss = alpha * sp * sg * sg
    ce_acc[...] += jnp.sum(loss, axis=0)

    matched = (cls != c).astype(jnp.float32)
    pb = pb_ref[...]
    tbx = tb_ref[...]
    l1 = jnp.sum(jnp.abs(pb - tbx), axis=1) * matched

    def corners(bx):
        cx = bx[:, 0, :]
        cy = bx[:, 1, :]
        w = bx[:, 2, :]
        h = bx[:, 3, :]
        return cx - 0.5 * w, cy - 0.5 * h, cx + 0.5 * w, cy + 0.5 * h

    ax0, ay0, ax1, ay1 = corners(pb)
    bx0, by0, bx1, by1 = corners(tbx)
    area_a = (ax1 - ax0) * (ay1 - ay0)
    area_b = (bx1 - bx0) * (by1 - by0)
    iw = jnp.maximum(jnp.minimum(ax1, bx1) - jnp.maximum(ax0, bx0), 0.0)
    ih = jnp.maximum(jnp.minimum(ay1, by1) - jnp.maximum(ay0, by0), 0.0)
    inter = iw * ih
    union = area_a + area_b - inter
    iou = inter / (union + 1e-7)
    ew = jnp.maximum(jnp.maximum(ax1, bx1) - jnp.minimum(ax0, bx0), 0.0)
    eh = jnp.maximum(jnp.maximum(ay1, by1) - jnp.minimum(ay0, by0), 0.0)
    area_e = ew * eh
    giou = iou - (area_e - union) / (area_e + 1e-7)

    m_acc[...] += matched
    l1_acc[...] += l1
    gi_acc[...] += (1.0 - giou) * matched

    @pl.when(b == nb - 1)
    def _final():
        s_ce = jnp.sum(ce_acc[...])
        s_m = jnp.sum(m_acc[...])
        s_l1 = jnp.sum(l1_acc[...])
        s_gi = jnp.sum(gi_acc[...])
        num_boxes = jnp.maximum(s_m, 1.0)
        lane128 = lax.broadcasted_iota(jnp.int32, (1, 128), 1)
        out_ref[...] = (s_ce * (lane128 == 0) + s_l1 * (lane128 == 1)
                        + s_gi * (lane128 == 2)) / num_boxes


def kernel(logits, pred_boxes, target_boxes, target_classes):
    B, Q, C = logits.shape
    xt = jnp.transpose(logits, (2, 0, 1))
    pbt = jnp.transpose(pred_boxes, (0, 2, 1))
    tbt = jnp.transpose(target_boxes, (0, 2, 1))
    cls = target_classes.astype(jnp.int32)
    grid = B // _BB
    outs = pl.pallas_call(
        _loss_block,
        grid=(grid,),
        in_specs=[
            pl.BlockSpec((C, _BB, Q), lambda b: (0, b, 0)),
            pl.BlockSpec((_BB, Q), lambda b: (b, 0)),
            pl.BlockSpec((_BB, 4, Q), lambda b: (b, 0, 0)),
            pl.BlockSpec((_BB, 4, Q), lambda b: (b, 0, 0)),
        ],
        out_specs=[
            pl.BlockSpec((_BB, Q), lambda b: (0, 0)),
            pl.BlockSpec((_BB, Q), lambda b: (0, 0)),
            pl.BlockSpec((_BB, Q), lambda b: (0, 0)),
            pl.BlockSpec((_BB, Q), lambda b: (0, 0)),
            pl.BlockSpec((1, 128), lambda b: (0, 0)),
        ],
        out_shape=[
            jax.ShapeDtypeStruct((_BB, Q), jnp.float32),
            jax.ShapeDtypeStruct((_BB, Q), jnp.float32),
            jax.ShapeDtypeStruct((_BB, Q), jnp.float32),
            jax.ShapeDtypeStruct((_BB, Q), jnp.float32),
            jax.ShapeDtypeStruct((1, 128), jnp.float32),
        ],
    )(xt, cls, pbt, tbt)
    return outs[4][0, :3]


# speedup vs baseline: 6.6457x; 1.3111x over previous
import jax
import jax.numpy as jnp
from jax import lax
from jax.experimental import pallas as pl

_BB = 8


def _loss_block(x_ref, cls_ref, pb_ref, tb_ref,
                ce_acc, m_acc, l1_acc, gi_acc, out_ref):
    b = pl.program_id(0)
    nb = pl.num_programs(0)

    @pl.when(b == 0)
    def _init():
        ce_acc[...] = jnp.zeros_like(ce_acc)
        m_acc[...] = jnp.zeros_like(m_acc)
        l1_acc[...] = jnp.zeros_like(l1_acc)
        gi_acc[...] = jnp.zeros_like(gi_acc)

    cls = cls_ref[...]
    c = x_ref.shape[0]

    def focal_chunk(base, cc):
        x = x_ref[pl.ds(base, cc), :, :]
        cidx = base + lax.broadcasted_iota(jnp.int32, x.shape, 0)
        t = cidx == cls[None, :, :]
        y = jnp.where(t, -x, x)
        e = jnp.exp(-jnp.abs(x))
        lp = jnp.log1p(e)
        s = 1.0 / (1.0 + e)
        sp = jnp.maximum(y, 0.0) + lp
        sg = jnp.where(y >= 0.0, s, 1.0 - s)
        alpha = jnp.where(t, 0.25, 0.75)
        loss = alpha * sp * sg * sg
        return jnp.sum(loss, axis=0)

    cc = 8
    part = focal_chunk(0, cc)
    for i in range(1, c // cc):
        part = part + focal_chunk(i * cc, cc)
    if c % cc:
        part = part + focal_chunk(c - c % cc, c % cc)
    ce_acc[...] += part

    matched = (cls != c).astype(jnp.float32)
    pb = pb_ref[...]
    tbx = tb_ref[...]
    l1 = jnp.sum(jnp.abs(pb - tbx), axis=1) * matched

    def corners(bx):
        cx = bx[:, 0, :]
        cy = bx[:, 1, :]
        w = bx[:, 2, :]
        h = bx[:, 3, :]
        return cx - 0.5 * w, cy - 0.5 * h, cx + 0.5 * w, cy + 0.5 * h

    ax0, ay0, ax1, ay1 = corners(pb)
    bx0, by0, bx1, by1 = corners(tbx)
    area_a = (ax1 - ax0) * (ay1 - ay0)
    area_b = (bx1 - bx0) * (by1 - by0)
    iw = jnp.maximum(jnp.minimum(ax1, bx1) - jnp.maximum(ax0, bx0), 0.0)
    ih = jnp.maximum(jnp.minimum(ay1, by1) - jnp.maximum(ay0, by0), 0.0)
    inter = iw * ih
    union = area_a + area_b - inter
    iou = inter / (union + 1e-7)
    ew = jnp.maximum(jnp.maximum(ax1, bx1) - jnp.minimum(ax0, bx0), 0.0)
    eh = jnp.maximum(jnp.maximum(ay1, by1) - jnp.minimum(ay0, by0), 0.0)
    area_e = ew * eh
    giou = iou - (area_e - union) / (area_e + 1e-7)

    m_acc[...] += matched
    l1_acc[...] += l1
    gi_acc[...] += (1.0 - giou) * matched

    @pl.when(b == nb - 1)
    def _final():
        s_ce = jnp.sum(ce_acc[...])
        s_m = jnp.sum(m_acc[...])
        s_l1 = jnp.sum(l1_acc[...])
        s_gi = jnp.sum(gi_acc[...])
        num_boxes = jnp.maximum(s_m, 1.0)
        lane128 = lax.broadcasted_iota(jnp.int32, (1, 128), 1)
        out_ref[...] = (s_ce * (lane128 == 0) + s_l1 * (lane128 == 1)
                        + s_gi * (lane128 == 2)) / num_boxes


def kernel(logits, pred_boxes, target_boxes, target_classes):
    B, Q, C = logits.shape
    xt = jnp.transpose(logits, (2, 0, 1))
    pbt = jnp.transpose(pred_boxes, (0, 2, 1))
    tbt = jnp.transpose(target_boxes, (0, 2, 1))
    cls = target_classes.astype(jnp.int32)
    grid = B // _BB
    outs = pl.pallas_call(
        _loss_block,
        grid=(grid,),
        in_specs=[
            pl.BlockSpec((C, _BB, Q), lambda b: (0, b, 0)),
            pl.BlockSpec((_BB, Q), lambda b: (b, 0)),
            pl.BlockSpec((_BB, 4, Q), lambda b: (b, 0, 0)),
            pl.BlockSpec((_BB, 4, Q), lambda b: (b, 0, 0)),
        ],
        out_specs=[
            pl.BlockSpec((_BB, Q), lambda b: (0, 0)),
            pl.BlockSpec((_BB, Q), lambda b: (0, 0)),
            pl.BlockSpec((_BB, Q), lambda b: (0, 0)),
            pl.BlockSpec((_BB, Q), lambda b: (0, 0)),
            pl.BlockSpec((1, 128), lambda b: (0, 0)),
        ],
        out_shape=[
            jax.ShapeDtypeStruct((_BB, Q), jnp.float32),
            jax.ShapeDtypeStruct((_BB, Q), jnp.float32),
            jax.ShapeDtypeStruct((_BB, Q), jnp.float32),
            jax.ShapeDtypeStruct((_BB, Q), jnp.float32),
            jax.ShapeDtypeStruct((1, 128), jnp.float32),
        ],
    )(xt, cls, pbt, tbt)
    return outs[4][0, :3]
